# Initial kernel scaffold; baseline (speedup 1.0000x reference)
import jax
import jax.numpy as jnp
from jax.experimental import pallas as pl


def kernel(indices, table):
    raise NotImplementedError("write your pallas kernel here")


# speedup vs baseline: 1.5008x; 1.5008x over previous
import functools

import jax
import jax.numpy as jnp
from jax import lax
from jax.experimental import pallas as pl
from jax.experimental.pallas import tpu as pltpu
from jax.experimental.pallas import tpu_sc as plsc

D = 32
NC, NS = 2, 16
NW = NC * NS
CHUNK = 128
K = 8


def _gather_rows(idx, table, b_per_w, n_chunks):
    n_outer = n_chunks // K
    rows_per_step = K * CHUNK
    mesh = plsc.VectorSubcoreMesh(core_axis_name="c", subcore_axis_name="s")

    @functools.partial(
        pl.kernel,
        out_type=jax.ShapeDtypeStruct((NW * b_per_w, D), jnp.float32),
        mesh=mesh,
        compiler_params=pltpu.CompilerParams(use_tc_tiling_on_sc=False),
        scratch_types=[
            pltpu.VMEM((n_chunks, CHUNK), jnp.int32),
            pltpu.VMEM((rows_per_step, D), jnp.float32),
            pltpu.VMEM((rows_per_step, D), jnp.float32),
            pltpu.SemaphoreType.DMA,
            pltpu.SemaphoreType.DMA,
            pltpu.SemaphoreType.DMA,
            pltpu.SemaphoreType.DMA,
        ],
    )
    def k(idx_hbm, table_hbm, out_hbm, idx_v, rows0, rows1, g0, g1, o0, o1):
        wid = lax.axis_index("s") * NC + lax.axis_index("c")
        base = wid * b_per_w
        pltpu.sync_copy(idx_hbm.at[wid], idx_v)
        rows = (rows0, rows1)
        gsem = (g0, g1)
        osem = (o0, o1)

        def fire(j, buf):
            for kk in range(K):
                pltpu.async_copy(
                    table_hbm.at[idx_v.at[j * K + kk]],
                    rows[buf].at[pl.ds(kk * CHUNK, CHUNK)],
                    gsem[buf],
                )

        def drain_gathers(j, buf):
            for kk in range(K):
                pltpu.make_async_copy(
                    table_hbm.at[idx_v.at[j * K + kk]],
                    rows[buf].at[pl.ds(kk * CHUNK, CHUNK)],
                    gsem[buf],
                ).wait()

        def out_copy(j, buf):
            pltpu.async_copy(
                rows[buf],
                out_hbm.at[pl.ds(base + j * rows_per_step, rows_per_step)],
                osem[buf],
            )

        def drain_out(j, buf):
            pltpu.make_async_copy(
                rows[buf],
                out_hbm.at[pl.ds(base + j * rows_per_step, rows_per_step)],
                osem[buf],
            ).wait()

        fire(0, 0)

        def body(j, _):
            buf = lax.rem(j, 2)
            for b in (0, 1):
                @pl.when(buf == b)
                def _():
                    @pl.when(j >= 2)
                    def _():
                        drain_out(j - 2, b)
                    fire(j, b)
                    drain_gathers(j - 1, 1 - b)
                    out_copy(j - 1, 1 - b)
            return 0

        lax.fori_loop(1, n_outer, body, 0, unroll=False)
        last = n_outer - 1
        lastbuf = last % 2
        if n_outer >= 2:
            drain_out(last - 1, 1 - lastbuf)
        drain_gathers(last, lastbuf)
        out_copy(last, lastbuf)
        drain_out(last, lastbuf)

    return k(idx, table)


def kernel(indices, table):
    B, S = indices.shape
    total = B * S
    b_per_w = total // NW
    n_chunks = b_per_w // CHUNK
    idx = indices.astype(jnp.int32).reshape(NW, n_chunks, CHUNK)
    out = _gather_rows(idx, table, b_per_w, n_chunks)
    return out.reshape(B, S, D)


# speedup vs baseline: 1.5016x; 1.0005x over previous
import functools

import jax
import jax.numpy as jnp
from jax import lax
from jax.experimental import pallas as pl
from jax.experimental.pallas import tpu as pltpu
from jax.experimental.pallas import tpu_sc as plsc

D = 32
NC, NS = 2, 16
NW = NC * NS
CHUNK = 256
K = 4


def _gather_rows(idx, table, b_per_w, n_chunks):
    n_outer = n_chunks // K
    rows_per_step = K * CHUNK
    mesh = plsc.VectorSubcoreMesh(core_axis_name="c", subcore_axis_name="s")

    @functools.partial(
        pl.kernel,
        out_type=jax.ShapeDtypeStruct((NW * b_per_w, D), jnp.float32),
        mesh=mesh,
        compiler_params=pltpu.CompilerParams(use_tc_tiling_on_sc=False),
        scratch_types=[
            pltpu.VMEM((n_chunks, CHUNK), jnp.int32),
            pltpu.VMEM((rows_per_step, D), jnp.float32),
            pltpu.VMEM((rows_per_step, D), jnp.float32),
            pltpu.SemaphoreType.DMA,
            pltpu.SemaphoreType.DMA,
            pltpu.SemaphoreType.DMA,
            pltpu.SemaphoreType.DMA,
        ],
    )
    def k(idx_hbm, table_hbm, out_hbm, idx_v, rows0, rows1, g0, g1, o0, o1):
        wid = lax.axis_index("s") * NC + lax.axis_index("c")
        base = wid * b_per_w
        pltpu.sync_copy(idx_hbm.at[wid], idx_v)
        rows = (rows0, rows1)
        gsem = (g0, g1)
        osem = (o0, o1)

        def fire(j, buf):
            for kk in range(K):
                pltpu.async_copy(
                    table_hbm.at[idx_v.at[j * K + kk]],
                    rows[buf].at[pl.ds(kk * CHUNK, CHUNK)],
                    gsem[buf],
                )

        def drain_gathers(j, buf):
            for kk in range(K):
                pltpu.make_async_copy(
                    table_hbm.at[idx_v.at[j * K + kk]],
                    rows[buf].at[pl.ds(kk * CHUNK, CHUNK)],
                    gsem[buf],
                ).wait()

        def out_copy(j, buf):
            pltpu.async_copy(
                rows[buf],
                out_hbm.at[pl.ds(base + j * rows_per_step, rows_per_step)],
                osem[buf],
            )

        def drain_out(j, buf):
            pltpu.make_async_copy(
                rows[buf],
                out_hbm.at[pl.ds(base + j * rows_per_step, rows_per_step)],
                osem[buf],
            ).wait()

        fire(0, 0)

        def body(j, _):
            buf = lax.rem(j, 2)
            for b in (0, 1):
                @pl.when(buf == b)
                def _():
                    @pl.when(j >= 2)
                    def _():
                        drain_out(j - 2, b)
                    fire(j, b)
                    drain_gathers(j - 1, 1 - b)
                    out_copy(j - 1, 1 - b)
            return 0

        lax.fori_loop(1, n_outer, body, 0, unroll=False)
        last = n_outer - 1
        lastbuf = last % 2
        if n_outer >= 2:
            drain_out(last - 1, 1 - lastbuf)
        drain_gathers(last, lastbuf)
        out_copy(last, lastbuf)
        drain_out(last, lastbuf)

    return k(idx, table)


def kernel(indices, table):
    B, S = indices.shape
    total = B * S
    b_per_w = total // NW
    n_chunks = b_per_w // CHUNK
    idx = indices.astype(jnp.int32).reshape(NW, n_chunks, CHUNK)
    out = _gather_rows(idx, table, b_per_w, n_chunks)
    return out.reshape(B, S, D)
